# Initial kernel scaffold; baseline (speedup 1.0000x reference)
import jax
import jax.numpy as jnp
from jax.experimental import pallas as pl


def kernel(x, edge_index, gate_logits, feat_mask):
    raise NotImplementedError("write your pallas kernel here")


# speedup vs baseline: 3.0682x; 3.0682x over previous
import math

import jax
import jax.numpy as jnp
from jax import lax
from jax.experimental import pallas as pl
from jax.experimental.pallas import tpu as pltpu
from jax.experimental.pallas import tpu_sc as plsc

N, E, D = 10000, 320000, 128
BETA = 1.0 / 3.0
GAMMA = -0.2
ZETA = 1.2
LOC_BIAS = 2.0
PEN_SHIFT = LOC_BIAS - BETA * math.log(-GAMMA / ZETA)

NC, NS, L = 2, 16, 16
CH = 128
NCHUNK = 160
EPT = NCHUNK * CH
EPAD = NS * EPT
HALF = D // 2
NP = 10240
RPT = NP // NS


def _sigmoid(v):
    return 1.0 / (1.0 + jnp.exp(-v))


def _body(x2, srcp, dstp, lgp, fm, out2, pen,
          src_v, lg_v, dst_v, fm_v, rows_v, pen_v, acc, sem):
    c = lax.axis_index("c")
    s = lax.axis_index("s")
    ebase = s * EPT

    pltpu.sync_copy(srcp.at[pl.ds(ebase, EPT)], src_v)
    pltpu.sync_copy(lgp.at[pl.ds(ebase, EPT)], lg_v)
    pltpu.sync_copy(dstp.at[pl.ds(s * NCHUNK, NCHUNK)], dst_v)
    pltpu.sync_copy(fm, fm_v)

    zero16 = jnp.zeros((L,), jnp.float32)

    def zrow(i, carry):
        for q in range(HALF // L):
            rows_v[i, pl.ds(q * L, L)] = zero16
        return carry

    lax.fori_loop(0, CH, zrow, 0)
    r0 = s * RPT
    for k in range(RPT // CH):
        pltpu.sync_copy(rows_v, acc.at[pl.ds(r0 + k * CH, CH)])

    def sfm(i, carry):
        sl = pl.ds(i * L, L)
        fm_v[sl] = _sigmoid(fm_v[sl])
        return carry

    lax.fori_loop(0, N // L, sfm, 0)

    def edge16(i, pacc):
        sl = pl.ds(i * L, L)
        sv = src_v[sl]
        lgv = lg_v[sl]
        fmg = plsc.load_gather(fm_v, [sv])
        gate = jnp.clip(_sigmoid(lgv + LOC_BIAS) * (ZETA - GAMMA) + GAMMA,
                        0.0, 1.0)
        lg_v[sl] = gate * fmg
        src_v[sl] = sv * 2 + c
        return pacc + _sigmoid(lgv + PEN_SHIFT)

    pen16 = lax.fori_loop(0, EPT // L, edge16, jnp.zeros((L,), jnp.float32))
    pen_v[...] = pen16
    pltpu.sync_copy(pen_v, pen.at[pl.ds((c * NS + s) * L, L)])

    plsc.subcore_barrier()

    def chunk(j, carry):
        pltpu.async_copy(x2.at[src_v.at[pl.ds(j * CH, CH)]], rows_v,
                         sem).wait()

        def grp(g, icarry):
            wv = lg_v[pl.ds(j * CH + g * L, L)]
            for k in range(L):
                e = g * L + k
                w = wv[k]
                for q in range(HALF // L):
                    sl = pl.ds(q * L, L)
                    rows_v[e, sl] = rows_v[e, sl] * w
            return icarry

        lax.fori_loop(0, CH // L, grp, 0)
        pltpu.sync_copy(rows_v, acc.at[dst_v.at[j]], add=True)
        return carry

    lax.fori_loop(0, NCHUNK, chunk, 0)

    plsc.subcore_barrier()

    pltpu.sync_copy(acc.at[pl.ds(r0, RPT)],
                    out2.at[pl.ds(c * NP + r0, RPT)])


_sc_call = pl.kernel(
    _body,
    out_type=(
        jax.ShapeDtypeStruct((NC * NP, HALF), jnp.float32),
        jax.ShapeDtypeStruct((NC * NS * L,), jnp.float32),
    ),
    mesh=plsc.VectorSubcoreMesh(core_axis_name="c", subcore_axis_name="s"),
    compiler_params=pltpu.CompilerParams(
        needs_layout_passes=False, use_tc_tiling_on_sc=False),
    scratch_types=[
        pltpu.VMEM((EPT,), jnp.int32),
        pltpu.VMEM((EPT,), jnp.float32),
        pltpu.VMEM((NCHUNK, CH), jnp.int32),
        pltpu.VMEM((N,), jnp.float32),
        pltpu.VMEM((CH, HALF), jnp.float32),
        pltpu.VMEM((L,), jnp.float32),
        pltpu.VMEM_SHARED((NP, HALF), jnp.float32),
        pltpu.SemaphoreType.DMA,
    ],
)


def kernel(x, edge_index, gate_logits, feat_mask):
    x2 = x.reshape(NC * N, HALF)
    pad = EPAD - E
    src = jnp.concatenate([edge_index[0], jnp.zeros((pad,), jnp.int32)])
    dst = jnp.concatenate([edge_index[1], jnp.zeros((pad,), jnp.int32)])
    lg = jnp.concatenate(
        [gate_logits, jnp.full((pad,), -1e30, jnp.float32)])
    dst2d = dst.reshape(EPAD // CH, CH)

    out2, pen = _sc_call(x2, src, dst2d, lg, feat_mask)
    out = jnp.concatenate([out2[:N], out2[NP:NP + N]], axis=1)
    penalty = jnp.sum(pen) / (NC * E)
    return out, penalty


# speedup vs baseline: 3.8678x; 1.2606x over previous
import math

import jax
import jax.numpy as jnp
from jax import lax
from jax.experimental import pallas as pl
from jax.experimental.pallas import tpu as pltpu
from jax.experimental.pallas import tpu_sc as plsc

N, E, D = 10000, 320000, 128
BETA = 1.0 / 3.0
GAMMA = -0.2
ZETA = 1.2
LOC_BIAS = 2.0
PEN_SHIFT = LOC_BIAS - BETA * math.log(-GAMMA / ZETA)

NC, NS, L = 2, 16, 16
CH = 128
NCHUNK = 160
EPT = NCHUNK * CH
EPAD = NS * EPT
HALF = D // 2
NP = 10240
RPT = NP // NS
NBUF = 4
A = 2


def _sigmoid(v):
    return 1.0 / (1.0 + jnp.exp(-v))


def _body(x2, srcp, dstp, lgp, fm, out2, pen,
          src_v, lg_v, dstb, fm_v, rows0, rows1, rows2, rows3, pen_v, acc,
          gs0, gs1, gs2, gs3, ss0, ss1, ss2, ss3):
    c = lax.axis_index("c")
    s = lax.axis_index("s")
    ebase = s * EPT
    rows = (rows0, rows1, rows2, rows3)
    gsems = (gs0, gs1, gs2, gs3)
    ssems = (ss0, ss1, ss2, ss3)

    pltpu.sync_copy(srcp.at[pl.ds(ebase, EPT)], src_v)
    pltpu.sync_copy(lgp.at[pl.ds(ebase, EPT)], lg_v)
    pltpu.sync_copy(fm, fm_v)

    zero16 = jnp.zeros((L,), jnp.float32)

    def zrow(i, carry):
        for q in range(HALF // L):
            rows0[i, pl.ds(q * L, L)] = zero16
        return carry

    lax.fori_loop(0, CH, zrow, 0)
    r0 = s * RPT
    for k in range(RPT // CH):
        pltpu.sync_copy(rows0, acc.at[pl.ds(r0 + k * CH, CH)])

    def sfm(i, carry):
        sl = pl.ds(i * L, L)
        fm_v[sl] = _sigmoid(fm_v[sl])
        return carry

    lax.fori_loop(0, N // L, sfm, 0)

    def edge16(i, pacc):
        sl = pl.ds(i * L, L)
        sv = src_v[sl]
        lgv = lg_v[sl]
        fmg = plsc.load_gather(fm_v, [sv])
        gate = jnp.clip(_sigmoid(lgv + LOC_BIAS) * (ZETA - GAMMA) + GAMMA,
                        0.0, 1.0)
        lg_v[sl] = gate * fmg
        src_v[sl] = sv * 2 + c
        return pacc + _sigmoid(lgv + PEN_SHIFT)

    pen16 = lax.fori_loop(0, EPT // L, edge16, jnp.zeros((L,), jnp.float32))
    pen_v[...] = pen16
    pltpu.sync_copy(pen_v, pen.at[pl.ds((c * NS + s) * L, L)])

    plsc.subcore_barrier()

    def issue_gather(j, b):
        pltpu.async_copy(x2.at[src_v.at[pl.ds(j * CH, CH)]], rows[b],
                         gsems[b])
        pltpu.async_copy(dstp.at[pl.ds(s * NCHUNK + j, 1)], dstb.at[b],
                         gsems[b])

    def wait_chunk(b):
        pltpu.make_async_copy(x2.at[pl.ds(0, CH)], rows[b],
                              gsems[b]).wait()
        pltpu.make_async_copy(dstp.at[pl.ds(0, 1)], dstb.at[b],
                              gsems[b]).wait()

    def wait_rows_dma(b, sem):
        pltpu.make_async_copy(x2.at[pl.ds(0, CH)], rows[b], sem).wait()

    for b in range(A):
        issue_gather(b, b)

    def step(t, carry):
        for b in range(NBUF):
            j = t * NBUF + b
            wait_chunk(b)

            def grp(g, icarry):
                wv = lg_v[pl.ds(j * CH + g * L, L)]
                for k in range(L):
                    e = g * L + k
                    w = wv[k]
                    for q in range(HALF // L):
                        sl = pl.ds(q * L, L)
                        rows[b][e, sl] = rows[b][e, sl] * w
                return icarry

            lax.fori_loop(0, CH // L, grp, 0, unroll=True)
            pltpu.async_copy(rows[b], acc.at[dstb.at[b, 0]], ssems[b],
                             add=True)

            bp = (b + A) % NBUF
            if b < A:
                @pl.when(t >= 1)
                def _drain():
                    wait_rows_dma(bp, ssems[bp])

                issue_gather(j + A, bp)
            else:
                wait_rows_dma(bp, ssems[bp])

                @pl.when(j + A <= NCHUNK - 1)
                def _refill():
                    issue_gather(j + A, bp)

        return carry

    lax.fori_loop(0, NCHUNK // NBUF, step, 0)

    for j in range(NCHUNK - A, NCHUNK):
        b = j % NBUF
        wait_rows_dma(b, ssems[b])

    plsc.subcore_barrier()

    pltpu.sync_copy(acc.at[pl.ds(r0, RPT)],
                    out2.at[pl.ds(c * NP + r0, RPT)])


_sc_call = pl.kernel(
    _body,
    out_type=(
        jax.ShapeDtypeStruct((NC * NP, HALF), jnp.float32),
        jax.ShapeDtypeStruct((NC * NS * L,), jnp.float32),
    ),
    mesh=plsc.VectorSubcoreMesh(core_axis_name="c", subcore_axis_name="s"),
    compiler_params=pltpu.CompilerParams(
        needs_layout_passes=False, use_tc_tiling_on_sc=False),
    scratch_types=[
        pltpu.VMEM((EPT,), jnp.int32),
        pltpu.VMEM((EPT,), jnp.float32),
        pltpu.VMEM((NBUF, 1, CH), jnp.int32),
        pltpu.VMEM((N,), jnp.float32),
        pltpu.VMEM((CH, HALF), jnp.float32),
        pltpu.VMEM((CH, HALF), jnp.float32),
        pltpu.VMEM((CH, HALF), jnp.float32),
        pltpu.VMEM((CH, HALF), jnp.float32),
        pltpu.VMEM((L,), jnp.float32),
        pltpu.VMEM_SHARED((NP, HALF), jnp.float32),
        pltpu.SemaphoreType.DMA,
        pltpu.SemaphoreType.DMA,
        pltpu.SemaphoreType.DMA,
        pltpu.SemaphoreType.DMA,
        pltpu.SemaphoreType.DMA,
        pltpu.SemaphoreType.DMA,
        pltpu.SemaphoreType.DMA,
        pltpu.SemaphoreType.DMA,
    ],
)


def kernel(x, edge_index, gate_logits, feat_mask):
    x2 = x.reshape(NC * N, HALF)
    pad = EPAD - E
    src = jnp.concatenate([edge_index[0], jnp.zeros((pad,), jnp.int32)])
    dst = jnp.concatenate([edge_index[1], jnp.zeros((pad,), jnp.int32)])
    lg = jnp.concatenate(
        [gate_logits, jnp.full((pad,), -1e30, jnp.float32)])
    dst2d = dst.reshape(EPAD // CH, CH)

    out2, pen = _sc_call(x2, src, dst2d, lg, feat_mask)
    out = jnp.concatenate([out2[:N], out2[NP:NP + N]], axis=1)
    penalty = jnp.sum(pen) / (NC * E)
    return out, penalty
